# baseline (speedup 1.0000x reference)
import functools

import jax
import jax.numpy as jnp
from jax import lax
from jax.experimental import pallas as pl
from jax.experimental.pallas import tpu as pltpu
from jax.experimental.pallas import tpu_sc as plsc

_B = 16384
_C = 1000

_info = plsc.get_sparse_core_info()
_NC = _info.num_cores
_NS = _info.num_subcores
_L = _info.num_lanes
_NW = _NC * _NS

_SC_COLS = 4096
_COLS_PER_W = _SC_COLS // _NW
_CHUNK_COLS = 128
_NCHUNK = _COLS_PER_W // _CHUNK_COLS

_TC_BLOCK = 1024
_TC_COLS = _B - _SC_COLS
_TC_BLK0 = _SC_COLS // _TC_BLOCK

_mesh = plsc.VectorSubcoreMesh(core_axis_name="c", subcore_axis_name="s")


@functools.partial(
    pl.kernel,
    mesh=_mesh,
    out_type=jax.ShapeDtypeStruct((_NW, _L), jnp.float32),
    scratch_types=[
        pltpu.VMEM((_COLS_PER_W,), jnp.int32),
        pltpu.VMEM((_C, _CHUNK_COLS), jnp.float32),
        pltpu.VMEM((_L,), jnp.float32),
        pltpu.SemaphoreType.DMA,
    ],
    compiler_params=pltpu.CompilerParams(use_tc_tiling_on_sc=True),
)
def _sc_partials(pred_hbm, label_hbm, out_hbm, lab_v, buf_v, acc_v, sem):
    wid = lax.axis_index("s") * _NC + lax.axis_index("c")
    base = wid * _COLS_PER_W

    pltpu.sync_copy(label_hbm.at[pl.ds(base, _COLS_PER_W)], lab_v)

    lane = lax.iota(jnp.int32, _L)
    acc = jnp.zeros((_L,), jnp.float32)
    for c in range(_NCHUNK):
        pltpu.async_copy(
            pred_hbm.at[:, pl.ds(base + c * _CHUNK_COLS, _CHUNK_COLS)],
            buf_v,
            sem,
        ).wait()

        def group(g, a):
            labs = lab_v[pl.ds(c * _CHUNK_COLS + g * _L, _L)]
            g16 = pl.multiple_of(g * _L, _L)
            for i in range(_L):
                v = buf_v[labs[i], pl.ds(g16, _L)]
                a = a + jnp.where(lane == i, v, jnp.float32(0.0))
            return a

        acc = lax.fori_loop(0, _CHUNK_COLS // _L, group, acc)

    acc_v[...] = acc
    pltpu.sync_copy(acc_v, out_hbm.at[wid])


def _tc_body(pred_ref, lab_ref, out_ref):
    i = pl.program_id(0)

    @pl.when(i == 0)
    def _():
        out_ref[0, 0] = jnp.float32(0.0)

    labs = lab_ref[0, 0, :]
    rows = lax.broadcasted_iota(jnp.int32, (_C, _TC_BLOCK), 0)
    sel = jnp.where(rows == labs[None, :], pred_ref[...], jnp.float32(0.0))
    out_ref[0, 0] += jnp.sum(sel)


_tc_sum = pl.pallas_call(
    _tc_body,
    grid=(_TC_COLS // _TC_BLOCK,),
    in_specs=[
        pl.BlockSpec((_C, _TC_BLOCK), lambda i: (0, _TC_BLK0 + i)),
        pl.BlockSpec((1, 1, _TC_BLOCK), lambda i: (_TC_BLK0 + i, 0, 0)),
    ],
    out_specs=pl.BlockSpec(
        (1, 1), lambda i: (0, 0), memory_space=pltpu.SMEM
    ),
    out_shape=jax.ShapeDtypeStruct((1, 1), jnp.float32),
    compiler_params=pltpu.CompilerParams(
        dimension_semantics=("arbitrary",)
    ),
)


def kernel(predict, label):
    pred_t = predict.T
    lab = label.astype(jnp.int32)
    sc_part = _sc_partials(pred_t, lab)
    tc_part = _tc_sum(pred_t, lab.reshape(_B // _TC_BLOCK, 1, _TC_BLOCK))
    return (sc_part.sum() + tc_part[0, 0]) / predict.shape[0]


# speedup vs baseline: 1.0094x; 1.0094x over previous
import functools

import jax
import jax.numpy as jnp
from jax import lax
from jax.experimental import pallas as pl
from jax.experimental.pallas import tpu as pltpu
from jax.experimental.pallas import tpu_sc as plsc

_B = 16384
_C = 1000

_info = plsc.get_sparse_core_info()
_NC = _info.num_cores
_NS = _info.num_subcores
_L = _info.num_lanes
_NW = _NC * _NS

_SC_COLS = 12288
_COLS_PER_W = _SC_COLS // _NW
_CHUNK_COLS = 128
_NCHUNK = _COLS_PER_W // _CHUNK_COLS

_TC_BLOCK = 1024
_TC_COLS = _B - _SC_COLS
_TC_BLK0 = _SC_COLS // _TC_BLOCK

_mesh = plsc.VectorSubcoreMesh(core_axis_name="c", subcore_axis_name="s")


@functools.partial(
    pl.kernel,
    mesh=_mesh,
    out_type=jax.ShapeDtypeStruct((_NW, _L), jnp.float32),
    scratch_types=[
        pltpu.VMEM((_COLS_PER_W,), jnp.int32),
        pltpu.VMEM((_C, _CHUNK_COLS), jnp.float32),
        pltpu.VMEM((_L,), jnp.float32),
        pltpu.SemaphoreType.DMA,
    ],
    compiler_params=pltpu.CompilerParams(use_tc_tiling_on_sc=True),
)
def _sc_partials(pred_hbm, label_hbm, out_hbm, lab_v, buf_v, acc_v, sem):
    wid = lax.axis_index("s") * _NC + lax.axis_index("c")
    base = wid * _COLS_PER_W

    pltpu.sync_copy(label_hbm.at[pl.ds(base, _COLS_PER_W)], lab_v)

    lane = lax.iota(jnp.int32, _L)
    acc = jnp.zeros((_L,), jnp.float32)
    for c in range(_NCHUNK):
        pltpu.async_copy(
            pred_hbm.at[:, pl.ds(base + c * _CHUNK_COLS, _CHUNK_COLS)],
            buf_v,
            sem,
        ).wait()

        def group(g, a):
            labs = lab_v[pl.ds(c * _CHUNK_COLS + g * _L, _L)]
            g16 = pl.multiple_of(g * _L, _L)
            for i in range(_L):
                v = buf_v[labs[i], pl.ds(g16, _L)]
                a = a + jnp.where(lane == i, v, jnp.float32(0.0))
            return a

        acc = lax.fori_loop(0, _CHUNK_COLS // _L, group, acc)

    acc_v[...] = acc
    pltpu.sync_copy(acc_v, out_hbm.at[wid])


def _tc_body(pred_ref, lab_ref, out_ref):
    i = pl.program_id(0)

    @pl.when(i == 0)
    def _():
        out_ref[0, 0] = jnp.float32(0.0)

    labs = lab_ref[0, 0, :]
    rows = lax.broadcasted_iota(jnp.int32, (_C, _TC_BLOCK), 0)
    sel = jnp.where(rows == labs[None, :], pred_ref[...], jnp.float32(0.0))
    out_ref[0, 0] += jnp.sum(sel)


_tc_sum = pl.pallas_call(
    _tc_body,
    grid=(_TC_COLS // _TC_BLOCK,),
    in_specs=[
        pl.BlockSpec((_C, _TC_BLOCK), lambda i: (0, _TC_BLK0 + i)),
        pl.BlockSpec((1, 1, _TC_BLOCK), lambda i: (_TC_BLK0 + i, 0, 0)),
    ],
    out_specs=pl.BlockSpec(
        (1, 1), lambda i: (0, 0), memory_space=pltpu.SMEM
    ),
    out_shape=jax.ShapeDtypeStruct((1, 1), jnp.float32),
    compiler_params=pltpu.CompilerParams(
        dimension_semantics=("arbitrary",)
    ),
)


def kernel(predict, label):
    pred_t = predict.T
    lab = label.astype(jnp.int32)
    sc_part = _sc_partials(pred_t, lab)
    tc_part = _tc_sum(pred_t, lab.reshape(_B // _TC_BLOCK, 1, _TC_BLOCK))
    return (sc_part.sum() + tc_part[0, 0]) / predict.shape[0]


# speedup vs baseline: 1.0184x; 1.0089x over previous
import functools

import jax
import jax.numpy as jnp
from jax import lax
from jax.experimental import pallas as pl
from jax.experimental.pallas import tpu as pltpu
from jax.experimental.pallas import tpu_sc as plsc

_B = 16384
_C = 1000

_info = plsc.get_sparse_core_info()
_NC = _info.num_cores
_NS = _info.num_subcores
_L = _info.num_lanes
_NW = _NC * _NS

_SC_COLS = 8192
_COLS_PER_W = _SC_COLS // _NW
_CHUNK_COLS = 128
_NCHUNK = _COLS_PER_W // _CHUNK_COLS

_TC_BLOCK = 512
_TC_COLS = _B - _SC_COLS
_TC_BLK0 = _SC_COLS // _TC_BLOCK

_mesh = plsc.VectorSubcoreMesh(core_axis_name="c", subcore_axis_name="s")


@functools.partial(
    pl.kernel,
    mesh=_mesh,
    out_type=jax.ShapeDtypeStruct((_NW, _L), jnp.float32),
    scratch_types=[
        pltpu.VMEM((_COLS_PER_W,), jnp.int32),
        pltpu.VMEM((_C, _CHUNK_COLS), jnp.float32),
        pltpu.VMEM((_L,), jnp.float32),
        pltpu.SemaphoreType.DMA,
    ],
    compiler_params=pltpu.CompilerParams(use_tc_tiling_on_sc=True),
)
def _sc_partials(pred_hbm, label_hbm, out_hbm, lab_v, buf_v, acc_v, sem):
    wid = lax.axis_index("s") * _NC + lax.axis_index("c")
    base = wid * _COLS_PER_W

    pltpu.sync_copy(label_hbm.at[pl.ds(base, _COLS_PER_W)], lab_v)

    lane = lax.iota(jnp.int32, _L)
    acc = jnp.zeros((_L,), jnp.float32)
    for c in range(_NCHUNK):
        pltpu.async_copy(
            pred_hbm.at[:, pl.ds(base + c * _CHUNK_COLS, _CHUNK_COLS)],
            buf_v,
            sem,
        ).wait()

        def group(g, a):
            labs = lab_v[pl.ds(c * _CHUNK_COLS + g * _L, _L)]
            g16 = pl.multiple_of(g * _L, _L)
            for i in range(_L):
                v = buf_v[labs[i], pl.ds(g16, _L)]
                a = a + jnp.where(lane == i, v, jnp.float32(0.0))
            return a

        acc = lax.fori_loop(0, _CHUNK_COLS // _L, group, acc)

    acc_v[...] = acc
    pltpu.sync_copy(acc_v, out_hbm.at[wid])


def _tc_body(pred_ref, lab_ref, out_ref):
    i = pl.program_id(0)

    @pl.when(i == 0)
    def _():
        out_ref[0, 0] = jnp.float32(0.0)

    labs = lab_ref[0, 0, :]
    rows = lax.broadcasted_iota(jnp.int32, (_C, _TC_BLOCK), 0)
    sel = jnp.where(rows == labs[None, :], pred_ref[...], jnp.float32(0.0))
    out_ref[0, 0] += jnp.sum(sel)


_tc_sum = pl.pallas_call(
    _tc_body,
    grid=(_TC_COLS // _TC_BLOCK,),
    in_specs=[
        pl.BlockSpec((_C, _TC_BLOCK), lambda i: (0, _TC_BLK0 + i)),
        pl.BlockSpec((1, 1, _TC_BLOCK), lambda i: (_TC_BLK0 + i, 0, 0)),
    ],
    out_specs=pl.BlockSpec(
        (1, 1), lambda i: (0, 0), memory_space=pltpu.SMEM
    ),
    out_shape=jax.ShapeDtypeStruct((1, 1), jnp.float32),
    compiler_params=pltpu.CompilerParams(
        dimension_semantics=("arbitrary",)
    ),
)


def kernel(predict, label):
    pred_t = predict.T
    lab = label.astype(jnp.int32)
    sc_part = _sc_partials(pred_t, lab)
    tc_part = _tc_sum(pred_t, lab.reshape(_B // _TC_BLOCK, 1, _TC_BLOCK))
    return (sc_part.sum() + tc_part[0, 0]) / predict.shape[0]


# speedup vs baseline: 1.0504x; 1.0315x over previous
import functools

import jax
import jax.numpy as jnp
from jax import lax
from jax.experimental import pallas as pl
from jax.experimental.pallas import tpu as pltpu
from jax.experimental.pallas import tpu_sc as plsc

_B = 16384
_C = 1000

_info = plsc.get_sparse_core_info()
_NC = _info.num_cores
_NS = _info.num_subcores
_L = _info.num_lanes
_NW = _NC * _NS

_SC_COLS = 8192
_COLS_PER_W = _SC_COLS // _NW
_CHUNK_COLS = 128
_NCHUNK = _COLS_PER_W // _CHUNK_COLS

_TC_BLOCK = 1024
_TC_COLS = _B - _SC_COLS
_TC_BLK0 = _SC_COLS // _TC_BLOCK

_mesh = plsc.VectorSubcoreMesh(core_axis_name="c", subcore_axis_name="s")


@functools.partial(
    pl.kernel,
    mesh=_mesh,
    out_type=jax.ShapeDtypeStruct((_NW, _L), jnp.float32),
    scratch_types=[
        pltpu.VMEM((_COLS_PER_W,), jnp.int32),
        pltpu.VMEM((_C, _CHUNK_COLS), jnp.float32),
        pltpu.VMEM((_L,), jnp.float32),
        pltpu.SemaphoreType.DMA,
    ],
    compiler_params=pltpu.CompilerParams(use_tc_tiling_on_sc=True),
)
def _sc_partials(pred_hbm, label_hbm, out_hbm, lab_v, buf_v, acc_v, sem):
    wid = lax.axis_index("s") * _NC + lax.axis_index("c")
    base = wid * _COLS_PER_W

    pltpu.sync_copy(label_hbm.at[pl.ds(base, _COLS_PER_W)], lab_v)

    lane = lax.iota(jnp.int32, _L)
    acc = jnp.zeros((_L,), jnp.float32)
    for c in range(_NCHUNK):
        pltpu.async_copy(
            pred_hbm.at[:, pl.ds(base + c * _CHUNK_COLS, _CHUNK_COLS)],
            buf_v,
            sem,
        ).wait()

        def group(g, a):
            labs = lab_v[pl.ds(c * _CHUNK_COLS + g * _L, _L)]
            g16 = pl.multiple_of(g * _L, _L)
            for i in range(_L):
                v = buf_v[labs[i], pl.ds(g16, _L)]
                a = a + jnp.where(lane == i, v, jnp.float32(0.0))
            return a

        acc = lax.fori_loop(0, _CHUNK_COLS // _L, group, acc)

    acc_v[...] = acc
    pltpu.sync_copy(acc_v, out_hbm.at[wid])


def _tc_body(pred_ref, lab_ref, out_ref):
    i = pl.program_id(0)

    @pl.when(i == 0)
    def _():
        out_ref[0, 0] = jnp.float32(0.0)

    labs = lab_ref[0, 0, :]
    rows = lax.broadcasted_iota(jnp.int32, (_C, _TC_BLOCK), 0)
    sel = jnp.where(rows == labs[None, :], pred_ref[...], jnp.float32(0.0))
    out_ref[0, 0] += jnp.sum(sel)


_tc_sum = pl.pallas_call(
    _tc_body,
    grid=(_TC_COLS // _TC_BLOCK,),
    in_specs=[
        pl.BlockSpec((_C, _TC_BLOCK), lambda i: (0, _TC_BLK0 + i)),
        pl.BlockSpec((1, 1, _TC_BLOCK), lambda i: (_TC_BLK0 + i, 0, 0)),
    ],
    out_specs=pl.BlockSpec(
        (1, 1), lambda i: (0, 0), memory_space=pltpu.SMEM
    ),
    out_shape=jax.ShapeDtypeStruct((1, 1), jnp.float32),
    compiler_params=pltpu.CompilerParams(
        dimension_semantics=("arbitrary",)
    ),
)


def kernel(predict, label):
    pred_t = predict.T
    lab = label.astype(jnp.int32)
    sc_part = _sc_partials(pred_t, lab)
    tc_part = _tc_sum(pred_t, lab.reshape(_B // _TC_BLOCK, 1, _TC_BLOCK))
    return (sc_part.sum() + tc_part[0, 0]) / predict.shape[0]
